# baseline (device time: 55461 ns/iter reference)
import jax
import jax.numpy as jnp
from jax import lax
from jax.experimental import pallas as pl
from jax.experimental.pallas import tpu as pltpu

N_DEV = 4
N_TOK = 1024
D_MODEL = 256
D_FF = 512
N_EXP = 16
EXP_PER_DEV = N_EXP // N_DEV
CHUNK = N_TOK // N_DEV


def kernel(x, router_W, route_idx, expert_W):
    def body(x_ref, rw_ref, idx_ref, ew_ref, out_ref, comm_ref, send_sems, recv_sems):
        my = lax.axis_index("i")
        left = lax.rem(my + N_DEV - 1, N_DEV)
        right = lax.rem(my + 1, N_DEV)

        barrier_sem = pltpu.get_barrier_semaphore()
        for nbr in (left, right):
            pl.semaphore_signal(
                barrier_sem, inc=1,
                device_id=(nbr,), device_id_type=pl.DeviceIdType.MESH,
            )
        pl.semaphore_wait(barrier_sem, 2)

        xv = x_ref[:, :]
        scores = jnp.dot(xv, rw_ref[:, :], preferred_element_type=jnp.float32)
        m = jnp.max(scores, axis=-1, keepdims=True)
        p = jnp.exp(scores - m)
        p = p / jnp.sum(p, axis=-1, keepdims=True)
        e0 = idx_ref[:, 0:1]
        e1 = idx_ref[:, 1:2]
        eids = lax.broadcasted_iota(jnp.int32, (N_TOK, N_EXP), 1)
        g0 = jnp.sum(jnp.where(eids == e0, p, 0.0), axis=-1, keepdims=True)
        g1 = jnp.sum(jnp.where(eids == e1, p, 0.0), axis=-1, keepdims=True)
        gs = g0 + g1

        partial = jnp.zeros((N_TOK, D_FF), jnp.float32)
        for j in range(EXP_PER_DEV):
            e = my * EXP_PER_DEV + j
            w = (jnp.where(e0 == e, g0, 0.0) + jnp.where(e1 == e, g1, 0.0)) / gs
            partial = partial + jnp.dot(
                xv * w, ew_ref[j], preferred_element_type=jnp.float32
            )
        out_ref[:, :] = partial

        for s in range(N_DEV - 1):
            c_send = lax.rem(my + N_DEV - s, N_DEV)
            rdma = pltpu.make_async_remote_copy(
                src_ref=out_ref.at[pl.ds(c_send * CHUNK, CHUNK), :],
                dst_ref=comm_ref.at[s],
                send_sem=send_sems.at[s],
                recv_sem=recv_sems.at[s],
                device_id=(right,),
                device_id_type=pl.DeviceIdType.MESH,
            )
            rdma.start()
            rdma.wait()
            c_recv = lax.rem(my + 2 * N_DEV - 1 - s, N_DEV)
            r0 = c_recv * CHUNK
            out_ref[pl.ds(r0, CHUNK), :] = (
                out_ref[pl.ds(r0, CHUNK), :] + comm_ref[s]
            )

        for s in range(N_DEV - 1):
            c = lax.rem(my + N_DEV + 1 - s, N_DEV)
            r0 = c * CHUNK
            rdma = pltpu.make_async_remote_copy(
                src_ref=out_ref.at[pl.ds(r0, CHUNK), :],
                dst_ref=out_ref.at[pl.ds(r0, CHUNK), :],
                send_sem=send_sems.at[N_DEV - 1 + s],
                recv_sem=recv_sems.at[N_DEV - 1 + s],
                device_id=(right,),
                device_id_type=pl.DeviceIdType.MESH,
            )
            rdma.start()
            rdma.wait()

    return pl.pallas_call(
        body,
        out_shape=jax.ShapeDtypeStruct((N_TOK, D_FF), jnp.float32),
        in_specs=[
            pl.BlockSpec(memory_space=pltpu.VMEM),
            pl.BlockSpec(memory_space=pltpu.VMEM),
            pl.BlockSpec(memory_space=pltpu.VMEM),
            pl.BlockSpec(memory_space=pltpu.VMEM),
        ],
        out_specs=pl.BlockSpec(memory_space=pltpu.VMEM),
        scratch_shapes=[
            pltpu.VMEM((N_DEV - 1, CHUNK, D_FF), jnp.float32),
            pltpu.SemaphoreType.DMA((2 * (N_DEV - 1),)),
            pltpu.SemaphoreType.DMA((2 * (N_DEV - 1),)),
        ],
        compiler_params=pltpu.CompilerParams(collective_id=0),
    )(x, router_W, route_idx, expert_W)


# device time: 39618 ns/iter; 1.3999x vs baseline; 1.3999x over previous
import jax
import jax.numpy as jnp
from jax import lax
from jax.experimental import pallas as pl
from jax.experimental.pallas import tpu as pltpu

N_DEV = 4
N_TOK = 1024
D_MODEL = 256
D_FF = 512
N_EXP = 16
EXP_PER_DEV = N_EXP // N_DEV
HALF_ROW = N_TOK // 2
QUAR_ROW = N_TOK // 4
HALF_COL = D_FF // 2


def kernel(x, router_W, route_idx, expert_W):
    def body(x_ref, rw_ref, idx_ref, ew_ref, out_ref,
             comm1_ref, comm2_ref, send_sems, recv_sems):
        my = lax.axis_index("i")
        x_c = (my >= 2).astype(jnp.int32)
        y_c = ((my == 1) | (my == 2)).astype(jnp.int32)
        p_y = my ^ 1
        p_x = 3 - my

        barrier_sem = pltpu.get_barrier_semaphore()
        for nbr in (p_y, p_x):
            pl.semaphore_signal(
                barrier_sem, inc=1,
                device_id=(nbr,), device_id_type=pl.DeviceIdType.MESH,
            )
        pl.semaphore_wait(barrier_sem, 2)

        xv = x_ref[:, :]
        scores = jnp.dot(xv, rw_ref[:, :], preferred_element_type=jnp.float32)
        m = jnp.max(scores, axis=-1, keepdims=True)
        p = jnp.exp(scores - m)
        p = p / jnp.sum(p, axis=-1, keepdims=True)
        e0 = idx_ref[:, 0:1]
        e1 = idx_ref[:, 1:2]
        eids = lax.broadcasted_iota(jnp.int32, (N_TOK, N_EXP), 1)
        g0 = jnp.sum(jnp.where(eids == e0, p, 0.0), axis=-1, keepdims=True)
        g1 = jnp.sum(jnp.where(eids == e1, p, 0.0), axis=-1, keepdims=True)
        gs = g0 + g1

        xw = []
        for j in range(EXP_PER_DEV):
            e = my * EXP_PER_DEV + j
            w = (jnp.where(e0 == e, g0, 0.0) + jnp.where(e1 == e, g1, 0.0)) / gs
            xw.append(xv * w)

        def chain_geom(ch):
            if ch == 0:
                v1, v2, prt1, prt2, c0 = y_c, x_c, p_y, p_x, 0
            else:
                v1, v2, prt1, prt2, c0 = x_c, y_c, p_x, p_y, HALF_COL
            keep1 = v1 * HALF_ROW
            send1 = (1 - v1) * HALF_ROW
            keep2 = keep1 + v2 * QUAR_ROW
            send2 = keep1 + (1 - v2) * QUAR_ROW
            return dict(v1=v1, v2=v2, prt=[prt1, prt2, prt2, prt1], c0=c0,
                        keep1=keep1, send1=send1, keep2=keep2, send2=send2)

        G = [chain_geom(0), chain_geom(1)]

        def make_rdma(ch, stage, src, dst):
            g = G[ch]
            k = ch * 4 + stage
            return pltpu.make_async_remote_copy(
                src_ref=src,
                dst_ref=dst,
                send_sem=send_sems.at[k],
                recv_sem=recv_sems.at[k],
                device_id=(g["prt"][stage],),
                device_id_type=pl.DeviceIdType.MESH,
            )

        def rs1(ch):
            g = G[ch]
            return make_rdma(
                ch, 0,
                out_ref.at[pl.ds(g["send1"], HALF_ROW), pl.ds(g["c0"], HALF_COL)],
                comm1_ref.at[ch],
            )

        def rs2(ch):
            g = G[ch]
            return make_rdma(
                ch, 1,
                out_ref.at[pl.ds(g["send2"], QUAR_ROW), pl.ds(g["c0"], HALF_COL)],
                comm2_ref.at[ch],
            )

        def ag1(ch):
            g = G[ch]
            sl = out_ref.at[pl.ds(g["keep2"], QUAR_ROW), pl.ds(g["c0"], HALF_COL)]
            return make_rdma(ch, 2, sl, sl)

        def ag2(ch):
            g = G[ch]
            sl = out_ref.at[pl.ds(g["keep1"], HALF_ROW), pl.ds(g["c0"], HALF_COL)]
            return make_rdma(ch, 3, sl, sl)

        def add1(ch):
            g = G[ch]
            r, c = pl.ds(g["keep1"], HALF_ROW), pl.ds(g["c0"], HALF_COL)
            out_ref[r, c] = out_ref[r, c] + comm1_ref[ch]

        def add2(ch):
            g = G[ch]
            r, c = pl.ds(g["keep2"], QUAR_ROW), pl.ds(g["c0"], HALF_COL)
            out_ref[r, c] = out_ref[r, c] + comm2_ref[ch]

        partialA = jnp.zeros((N_TOK, HALF_COL), jnp.float32)
        for j in range(EXP_PER_DEV):
            partialA = partialA + jnp.dot(
                xw[j], ew_ref[j, :, 0:HALF_COL],
                preferred_element_type=jnp.float32,
            )
        out_ref[:, 0:HALF_COL] = partialA
        a = rs1(0)
        a.start()

        partialB = jnp.zeros((N_TOK, HALF_COL), jnp.float32)
        for j in range(EXP_PER_DEV):
            partialB = partialB + jnp.dot(
                xw[j], ew_ref[j, :, HALF_COL:D_FF],
                preferred_element_type=jnp.float32,
            )
        out_ref[:, HALF_COL:D_FF] = partialB
        b = rs1(1)
        b.start()

        a.wait()
        add1(0)
        a = rs2(0)
        a.start()

        b.wait()
        add1(1)
        b = rs2(1)
        b.start()

        a.wait()
        add2(0)
        a = ag1(0)
        a.start()

        b.wait()
        add2(1)
        b = ag1(1)
        b.start()

        a.wait()
        a = ag2(0)
        a.start()

        b.wait()
        b = ag2(1)
        b.start()

        a.wait()
        b.wait()

    return pl.pallas_call(
        body,
        out_shape=jax.ShapeDtypeStruct((N_TOK, D_FF), jnp.float32),
        in_specs=[
            pl.BlockSpec(memory_space=pltpu.VMEM),
            pl.BlockSpec(memory_space=pltpu.VMEM),
            pl.BlockSpec(memory_space=pltpu.VMEM),
            pl.BlockSpec(memory_space=pltpu.VMEM),
        ],
        out_specs=pl.BlockSpec(memory_space=pltpu.VMEM),
        scratch_shapes=[
            pltpu.VMEM((2, HALF_ROW, HALF_COL), jnp.float32),
            pltpu.VMEM((2, QUAR_ROW, HALF_COL), jnp.float32),
            pltpu.SemaphoreType.DMA((8,)),
            pltpu.SemaphoreType.DMA((8,)),
        ],
        compiler_params=pltpu.CompilerParams(collective_id=0),
    )(x, router_W, route_idx, expert_W)


# device time: 27027 ns/iter; 2.0521x vs baseline; 1.4659x over previous
import jax
import jax.numpy as jnp
from jax import lax
from jax.experimental import pallas as pl
from jax.experimental.pallas import tpu as pltpu

N_DEV = 4
N_TOK = 1024
D_MODEL = 256
D_FF = 512
N_EXP = 16
EXP_PER_DEV = N_EXP // N_DEV
HALF_ROW = N_TOK // 2
QUAR_ROW = N_TOK // 4
HALF_COL = D_FF // 2
BF = jnp.bfloat16
F32 = jnp.float32


def kernel(x, router_W, route_idx, expert_W):
    def body(x_ref, rw_ref, idx_ref, ew_ref, out_ref,
             comm1_ref, comm2_ref, sbuf1_ref, sbuf2_ref, hbuf_ref, obuf_ref,
             send_sems, recv_sems):
        my = lax.axis_index("i")
        x_c = (my >= 2).astype(jnp.int32)
        y_c = ((my == 1) | (my == 2)).astype(jnp.int32)
        p_y = my ^ 1
        p_x = 3 - my

        barrier_sem = pltpu.get_barrier_semaphore()
        for nbr in (p_y, p_x):
            pl.semaphore_signal(
                barrier_sem, inc=1,
                device_id=(nbr,), device_id_type=pl.DeviceIdType.MESH,
            )
        pl.semaphore_wait(barrier_sem, 2)

        xv = x_ref[:, :]
        scores = jnp.dot(xv, rw_ref[:, :], preferred_element_type=F32)
        m = jnp.max(scores, axis=-1, keepdims=True)
        p = jnp.exp(scores - m)
        p = p / jnp.sum(p, axis=-1, keepdims=True)
        e0 = idx_ref[:, 0:1]
        e1 = idx_ref[:, 1:2]
        eids = lax.broadcasted_iota(jnp.int32, (N_TOK, N_EXP), 1)
        g0 = jnp.sum(jnp.where(eids == e0, p, 0.0), axis=-1, keepdims=True)
        g1 = jnp.sum(jnp.where(eids == e1, p, 0.0), axis=-1, keepdims=True)
        gs = g0 + g1

        xw = []
        for j in range(EXP_PER_DEV):
            e = my * EXP_PER_DEV + j
            w = (jnp.where(e0 == e, g0, 0.0) + jnp.where(e1 == e, g1, 0.0)) / gs
            xw.append((xv * w).astype(BF))

        if True:
            v1a, v2a = y_c, x_c
            v1b, v2b = x_c, y_c
        G = [
            dict(v1=v1a, v2=v2a, prt=[p_y, p_x, p_x, p_y], c0=0),
            dict(v1=v1b, v2=v2b, prt=[p_x, p_y, p_y, p_x], c0=HALF_COL),
        ]
        for g in G:
            g["keep1"] = g["v1"] * HALF_ROW
            g["send1"] = (1 - g["v1"]) * HALF_ROW
            g["keep2"] = g["keep1"] + g["v2"] * QUAR_ROW
            g["send2"] = g["keep1"] + (1 - g["v2"]) * QUAR_ROW
            g["off2"] = g["v2"] * QUAR_ROW
            g["roff2"] = (1 - g["v2"]) * QUAR_ROW

        def make_rdma(ch, stage, src, dst):
            k = ch * 4 + stage
            return pltpu.make_async_remote_copy(
                src_ref=src,
                dst_ref=dst,
                send_sem=send_sems.at[k],
                recv_sem=recv_sems.at[k],
                device_id=(G[ch]["prt"][stage],),
                device_id_type=pl.DeviceIdType.MESH,
            )

        def cols(ch):
            return pl.ds(G[ch]["c0"], HALF_COL)

        def rs1_start(ch):
            g = G[ch]
            sbuf1_ref[ch, :, :] = out_ref[pl.ds(g["send1"], HALF_ROW), cols(ch)].astype(BF)
            r = make_rdma(ch, 0, sbuf1_ref.at[ch], comm1_ref.at[ch])
            r.start()
            return r

        def rs1_fin(ch, r):
            g = G[ch]
            r.wait()
            sl = (pl.ds(g["keep1"], HALF_ROW), cols(ch))
            out_ref[sl] = out_ref[sl] + comm1_ref[ch].astype(F32)

        def rs2_start(ch):
            g = G[ch]
            sbuf2_ref[ch, :, :] = out_ref[pl.ds(g["send2"], QUAR_ROW), cols(ch)].astype(BF)
            r = make_rdma(ch, 1, sbuf2_ref.at[ch], comm2_ref.at[ch])
            r.start()
            return r

        def rs2_fin(ch, r):
            g = G[ch]
            r.wait()
            sl = (pl.ds(g["keep2"], QUAR_ROW), cols(ch))
            q = out_ref[sl] + comm2_ref[ch].astype(F32)
            out_ref[sl] = q
            hbuf_ref[ch, pl.ds(g["off2"], QUAR_ROW), :] = q.astype(BF)

        def ag1_start(ch):
            g = G[ch]
            sl = hbuf_ref.at[ch, pl.ds(g["off2"], QUAR_ROW), :]
            r = make_rdma(ch, 2, sl, sl)
            r.start()
            return r

        def ag1_fin(ch, r):
            g = G[ch]
            r.wait()
            out_ref[pl.ds(g["keep1"] + g["roff2"], QUAR_ROW), cols(ch)] = (
                hbuf_ref[ch, pl.ds(g["roff2"], QUAR_ROW), :].astype(F32)
            )

        def ag2_start(ch):
            r = make_rdma(ch, 3, hbuf_ref.at[ch], obuf_ref.at[ch])
            r.start()
            return r

        def ag2_fin(ch, r):
            g = G[ch]
            r.wait()
            out_ref[pl.ds(g["send1"], HALF_ROW), cols(ch)] = (
                obuf_ref[ch].astype(F32)
            )

        ewb = ew_ref[:, :, :].astype(BF)
        partialA = jnp.zeros((N_TOK, HALF_COL), F32)
        for j in range(EXP_PER_DEV):
            partialA = partialA + jnp.dot(
                xw[j], ewb[j, :, 0:HALF_COL], preferred_element_type=F32
            )
        out_ref[:, 0:HALF_COL] = partialA
        a = rs1_start(0)

        partialB = jnp.zeros((N_TOK, HALF_COL), F32)
        for j in range(EXP_PER_DEV):
            partialB = partialB + jnp.dot(
                xw[j], ewb[j, :, HALF_COL:D_FF], preferred_element_type=F32
            )
        out_ref[:, HALF_COL:D_FF] = partialB
        b = rs1_start(1)

        rs1_fin(0, a)
        a = rs2_start(0)
        rs1_fin(1, b)
        b = rs2_start(1)

        rs2_fin(0, a)
        a = ag1_start(0)
        rs2_fin(1, b)
        b = ag1_start(1)

        ag1_fin(0, a)
        a = ag2_start(0)
        ag1_fin(1, b)
        b = ag2_start(1)

        ag2_fin(0, a)
        ag2_fin(1, b)

    return pl.pallas_call(
        body,
        out_shape=jax.ShapeDtypeStruct((N_TOK, D_FF), F32),
        in_specs=[
            pl.BlockSpec(memory_space=pltpu.VMEM),
            pl.BlockSpec(memory_space=pltpu.VMEM),
            pl.BlockSpec(memory_space=pltpu.VMEM),
            pl.BlockSpec(memory_space=pltpu.VMEM),
        ],
        out_specs=pl.BlockSpec(memory_space=pltpu.VMEM),
        scratch_shapes=[
            pltpu.VMEM((2, HALF_ROW, HALF_COL), BF),
            pltpu.VMEM((2, QUAR_ROW, HALF_COL), BF),
            pltpu.VMEM((2, HALF_ROW, HALF_COL), BF),
            pltpu.VMEM((2, QUAR_ROW, HALF_COL), BF),
            pltpu.VMEM((2, HALF_ROW, HALF_COL), BF),
            pltpu.VMEM((2, HALF_ROW, HALF_COL), BF),
            pltpu.SemaphoreType.DMA((8,)),
            pltpu.SemaphoreType.DMA((8,)),
        ],
        compiler_params=pltpu.CompilerParams(collective_id=0),
    )(x, router_W, route_idx, expert_W)
